# baseline (speedup 1.0000x reference)
import functools

import jax
import jax.numpy as jnp
import numpy as np
from jax import lax
from jax.experimental import pallas as pl
from jax.experimental.pallas import tpu as pltpu
from jax.experimental.pallas import tpu_sc as plsc

B = 4096
T = 16
D = 3
N_CODES = 1024
CODE_DIM = 32
N_TOKENS = 8
HIDDEN = 256

BB = 1024
N_TOK_TOTAL = B * N_TOKENS
NW = 32
TOK_PER_W = N_TOK_TOTAL // NW


def _encode_kernel(x_ref, w1_ref, b1_ref, w2_ref, b2_ref, cbt_ref,
                   cbsq_ref, mavg_ref, idx_ref):
    x = x_ref[...]
    h = jnp.maximum(
        jnp.dot(x, w1_ref[...], preferred_element_type=jnp.float32)
        + b1_ref[...], 0.0)
    z = jnp.dot(h, w2_ref[...], preferred_element_type=jnp.float32) + b2_ref[...]

    mavg = mavg_ref[...]
    mu = jnp.dot(z, mavg, preferred_element_type=jnp.float32,
                 precision=jax.lax.Precision.HIGHEST)
    zc = z - mu
    var = jnp.dot(zc * zc, mavg, preferred_element_type=jnp.float32,
                  precision=jax.lax.Precision.HIGHEST)
    ze = zc * jax.lax.rsqrt(var + 1e-5)

    cbt = cbt_ref[...]
    cbsq = cbsq_ref[...]

    ds = []
    for t in range(N_TOKENS):
        zet = ze[:, t * CODE_DIM:(t + 1) * CODE_DIM]
        ds.append(cbsq - 2.0 * jnp.dot(zet, cbt,
                                       preferred_element_type=jnp.float32))
    ms = [jnp.min(d, axis=1, keepdims=True) for d in ds]
    lanes = jax.lax.broadcasted_iota(jnp.int32, (BB, N_CODES), 1)
    idx_cols = []
    for t in range(N_TOKENS):
        masked = jnp.where(ds[t] <= ms[t], lanes, N_CODES)
        idx_cols.append(jnp.min(masked, axis=1)[:, None])
    idx_ref[...] = jnp.concatenate(idx_cols, axis=1)


def _decode_kernel(zq_ref, w3_ref, b3_ref, w4_ref, b4_ref, recon_ref):
    zq = zq_ref[...]
    h2 = jnp.maximum(
        jnp.dot(zq, w3_ref[...], preferred_element_type=jnp.float32)
        + b3_ref[...], 0.0)
    recon_ref[...] = (
        jnp.dot(h2, w4_ref[...], preferred_element_type=jnp.float32)
        + b4_ref[...])


def _sc_gather(codebook, idx_flat):
    mesh = plsc.VectorSubcoreMesh(core_axis_name="c", subcore_axis_name="s")

    @functools.partial(
        pl.kernel, mesh=mesh,
        compiler_params=pltpu.CompilerParams(use_tc_tiling_on_sc=False),
        out_type=jax.ShapeDtypeStruct((N_TOK_TOTAL, CODE_DIM), jnp.float32),
        scratch_types=[
            pltpu.VMEM((TOK_PER_W,), jnp.int32),
            pltpu.VMEM((TOK_PER_W, CODE_DIM), jnp.float32),
            pltpu.SemaphoreType.DMA,
        ],
    )
    def gather_k(table_hbm, idx_hbm, out_hbm, idx_v, rows_v, sem):
        wid = lax.axis_index("s") * 2 + lax.axis_index("c")
        base = wid * TOK_PER_W
        pltpu.sync_copy(idx_hbm.at[pl.ds(base, TOK_PER_W)], idx_v)
        pltpu.async_copy(table_hbm.at[idx_v], rows_v, sem).wait()
        pltpu.sync_copy(rows_v, out_hbm.at[pl.ds(base, TOK_PER_W)])

    return gather_k(codebook, idx_flat)


@jax.jit
def _run(x2, W1, b1, W2, b2, codebook, W3, b3, W4, b4):
    cbt = codebook.T
    cbsq = jnp.sum(codebook * codebook, axis=1)[None, :]
    mavg = jnp.asarray(
        np.kron(np.eye(N_TOKENS, dtype=np.float32),
                np.full((CODE_DIM, CODE_DIM), 1.0 / CODE_DIM,
                        dtype=np.float32)))
    grid = (B // BB,)

    def bspec(shape):
        return pl.BlockSpec(shape, lambda i: (0,) * len(shape))

    idx = pl.pallas_call(
        _encode_kernel,
        grid=grid,
        in_specs=[
            pl.BlockSpec((BB, T * D), lambda i: (i, 0)),
            bspec((T * D, HIDDEN)),
            bspec((1, HIDDEN)),
            bspec((HIDDEN, N_TOKENS * CODE_DIM)),
            bspec((1, N_TOKENS * CODE_DIM)),
            bspec((CODE_DIM, N_CODES)),
            bspec((1, N_CODES)),
            bspec((N_TOKENS * CODE_DIM, N_TOKENS * CODE_DIM)),
        ],
        out_specs=pl.BlockSpec((BB, N_TOKENS), lambda i: (i, 0)),
        out_shape=jax.ShapeDtypeStruct((B, N_TOKENS), jnp.int32),
        compiler_params=pltpu.CompilerParams(
            dimension_semantics=("arbitrary",)),
    )(x2, W1, b1[None, :], W2, b2[None, :], cbt, cbsq, mavg)

    zq_flat = _sc_gather(codebook, idx.reshape(N_TOK_TOTAL))
    zq2 = zq_flat.reshape(B, N_TOKENS * CODE_DIM)

    recon = pl.pallas_call(
        _decode_kernel,
        grid=grid,
        in_specs=[
            pl.BlockSpec((BB, N_TOKENS * CODE_DIM), lambda i: (i, 0)),
            bspec((N_TOKENS * CODE_DIM, HIDDEN)),
            bspec((1, HIDDEN)),
            bspec((HIDDEN, T * D)),
            bspec((1, T * D)),
        ],
        out_specs=pl.BlockSpec((BB, T * D), lambda i: (i, 0)),
        out_shape=jax.ShapeDtypeStruct((B, T * D), jnp.float32),
        compiler_params=pltpu.CompilerParams(
            dimension_semantics=("arbitrary",)),
    )(zq2, W3, b3[None, :], W4, b4[None, :])
    return recon, zq2, idx


def kernel(x, W1, b1, W2, b2, codebook, W3, b3, W4, b4):
    x2 = x.reshape(B, T * D)
    recon, zq, idx = _run(x2, W1, b1, W2, b2, codebook, W3, b3, W4, b4)
    return (recon.reshape(B, T, D),
            zq.reshape(B, N_TOKENS, CODE_DIM),
            idx)
